# baseline (speedup 1.0000x reference)
import functools

import jax
import jax.numpy as jnp
from jax import lax
from jax.experimental import pallas as pl
from jax.experimental.pallas import tpu as pltpu
from jax.experimental.pallas import tpu_sc as plsc

N = 10000
D = 128
E = 320000
NC = 2
NS = 16
NW = NC * NS
NP = 10240
SLAB = NP // NS
EP = E // NW
C = 128
NFULL = EP // C
TAIL = EP - NFULL * C
BM = 256

_MESH = plsc.VectorSubcoreMesh(core_axis_name="c", subcore_axis_name="s")


def _sc_agg_body(h_hbm, src_hbm, dst_hbm, z128_hbm, out_hbm,
                 acc, src_v, dst_v, rows_v, src_t, dst_t, rows_t, gsem):
    c = lax.axis_index("c")
    s = lax.axis_index("s")
    wid = c * NS + s
    base = wid * EP
    row0 = s * SLAB

    pltpu.sync_copy(z128_hbm, rows_v)
    for k in range(SLAB // C):
        pltpu.sync_copy(rows_v, acc.at[pl.ds(row0 + k * C, C)])
    plsc.subcore_barrier()

    @pl.loop(0, NFULL)
    def chunk_body(i):
        off = base + i * C
        pltpu.sync_copy(src_hbm.at[pl.ds(off, C)], src_v)
        pltpu.sync_copy(dst_hbm.at[pl.ds(off, C)], dst_v)
        pltpu.async_copy(h_hbm.at[src_v], rows_v, gsem).wait()
        pltpu.sync_copy(rows_v, acc.at[dst_v], add=True)

    offt = base + NFULL * C
    pltpu.sync_copy(src_hbm.at[pl.ds(offt, TAIL)], src_t)
    pltpu.sync_copy(dst_hbm.at[pl.ds(offt, TAIL)], dst_t)
    pltpu.async_copy(h_hbm.at[src_t], rows_t, gsem).wait()
    pltpu.sync_copy(rows_t, acc.at[dst_t], add=True)

    plsc.subcore_barrier()

    for k in range(SLAB // C):
        pltpu.sync_copy(acc.at[pl.ds(row0 + k * C, C)], rows_v)
        pltpu.sync_copy(rows_v, out_hbm.at[pl.ds(c * NP + row0 + k * C, C)])


_sc_agg = functools.partial(
    pl.kernel, mesh=_MESH,
    out_type=jax.ShapeDtypeStruct((NC * NP, D), jnp.float32),
    scratch_types=[
        pltpu.VMEM_SHARED((NP, D), jnp.float32),
        pltpu.VMEM((C,), jnp.int32),
        pltpu.VMEM((C,), jnp.int32),
        pltpu.VMEM((C, D), jnp.float32),
        pltpu.VMEM((TAIL,), jnp.int32),
        pltpu.VMEM((TAIL,), jnp.int32),
        pltpu.VMEM((TAIL, D), jnp.float32),
        pltpu.SemaphoreType.DMA,
    ])(_sc_agg_body)


def _dense_body(agg0, agg1, cnt0, cnt1, h, wl, wr, b, out):
    a = agg0[...] + agg1[...]
    cnt = cnt0[...][:, 0:1] + cnt1[...][:, 0:1]
    mean = a * (1.0 / jnp.maximum(cnt, 1.0))
    acc = lax.dot_general(mean, wl[...], (((1,), (1,)), ((), ())),
                          preferred_element_type=jnp.float32)
    acc = acc + lax.dot_general(h[...], wr[...], (((1,), (1,)), ((), ())),
                                preferred_element_type=jnp.float32)
    acc = acc + b[...]
    out[...] = jnp.maximum(acc, 0.0)


def _dense_final_body(agg0, agg1, cnt0, cnt1, h, wl, wr, b, fcw, fcb, out):
    a = agg0[...] + agg1[...]
    cnt = cnt0[...][:, 0:1] + cnt1[...][:, 0:1]
    mean = a * (1.0 / jnp.maximum(cnt, 1.0))
    acc = lax.dot_general(mean, wl[...], (((1,), (1,)), ((), ())),
                          preferred_element_type=jnp.float32)
    acc = acc + lax.dot_general(h[...], wr[...], (((1,), (1,)), ((), ())),
                                preferred_element_type=jnp.float32)
    acc = jnp.maximum(acc + b[...], 0.0)
    out[...] = lax.dot_general(acc, fcw[...], (((1,), (1,)), ((), ())),
                               preferred_element_type=jnp.float32) + fcb[...]


def _row_spec(i):
    return (i, 0)


def _row_spec_hi(i):
    return (i + NP // BM, 0)


def _fixed_spec(i):
    return (0, 0)


_COMMON_SPECS = [
    pl.BlockSpec((BM, D), _row_spec),
    pl.BlockSpec((BM, D), _row_spec_hi),
    pl.BlockSpec((BM, D), _row_spec),
    pl.BlockSpec((BM, D), _row_spec_hi),
    pl.BlockSpec((BM, D), _row_spec),
    pl.BlockSpec((D, D), _fixed_spec),
    pl.BlockSpec((D, D), _fixed_spec),
    pl.BlockSpec((1, D), _fixed_spec),
]

_dense = pl.pallas_call(
    _dense_body,
    grid=(NP // BM,),
    in_specs=_COMMON_SPECS,
    out_specs=pl.BlockSpec((BM, D), _row_spec),
    out_shape=jax.ShapeDtypeStruct((NP, D), jnp.float32),
)

_dense_final = pl.pallas_call(
    _dense_final_body,
    grid=(NP // BM,),
    in_specs=_COMMON_SPECS + [
        pl.BlockSpec((D, D), _fixed_spec),
        pl.BlockSpec((1, D), _fixed_spec),
    ],
    out_specs=pl.BlockSpec((BM, D), _row_spec),
    out_shape=jax.ShapeDtypeStruct((NP, D), jnp.float32),
)


def kernel(x, edge_index, Wl1a, bl1a, Wr1a, Wl1b, bl1b, Wr1b,
           Wl2a, bl2a, Wr2a, Wl2b, bl2b, Wr2b, fc_W, fc_b):
    src = edge_index[0]
    dst = edge_index[1]
    xp = jnp.pad(x, ((0, NP - N), (0, 0)))
    z128 = jnp.zeros((C, D), jnp.float32)
    ones_tbl = jnp.ones((8, D), jnp.float32)
    zidx = jnp.zeros((E,), jnp.int32)

    cnt = _sc_agg(ones_tbl, zidx, dst, z128)
    agg = _sc_agg(xp, src, dst, z128)
    h = _dense(agg, agg, cnt, cnt, xp, Wl1a, Wr1a, bl1a.reshape(1, D))
    agg2 = _sc_agg(h, src, dst, z128)
    h = _dense(agg2, agg2, cnt, cnt, h, Wl1b, Wr1b, bl1b.reshape(1, D))
    agg3 = _sc_agg(h, src, dst, z128)
    h = _dense(agg3, agg3, cnt, cnt, h, Wl2a, Wr2a, bl2a.reshape(1, D))
    agg4 = _sc_agg(h, src, dst, z128)
    h = _dense_final(agg4, agg4, cnt, cnt, h, Wl2b, Wr2b, bl2b.reshape(1, D),
                     fc_W, fc_b.reshape(1, D))
    return h[:N]


# speedup vs baseline: 9.5671x; 9.5671x over previous
import functools

import jax
import jax.numpy as jnp
from jax import lax
from jax.experimental import pallas as pl
from jax.experimental.pallas import tpu as pltpu
from jax.experimental.pallas import tpu_sc as plsc

N = 10000
D = 128
E = 320000
NC = 2
NS = 16
NW = NC * NS
NP = 10240
SLAB = NP // NS
EP = E // NW
C = 128
NFULL = EP // C
TAIL = EP - NFULL * C
BM = 256

_MESH = plsc.VectorSubcoreMesh(core_axis_name="c", subcore_axis_name="s")


def _sc_agg_body(h_hbm, src_hbm, dst_hbm, z128_hbm, out_hbm,
                 acc, src_v, dst_v, rows_v, src_t, dst_t, rows_t, gsem):
    c = lax.axis_index("c")
    s = lax.axis_index("s")
    wid = c * NS + s
    base = wid * EP
    row0 = s * SLAB

    pltpu.sync_copy(z128_hbm, rows_v)
    for k in range(SLAB // C):
        pltpu.sync_copy(rows_v, acc.at[pl.ds(row0 + k * C, C)])
    plsc.subcore_barrier()

    @pl.loop(0, NFULL)
    def chunk_body(i):
        off = base + i * C
        pltpu.sync_copy(src_hbm.at[pl.ds(off, C)], src_v)
        pltpu.sync_copy(dst_hbm.at[pl.ds(off, C)], dst_v)
        pltpu.async_copy(h_hbm.at[src_v], rows_v, gsem).wait()
        pltpu.sync_copy(rows_v, acc.at[dst_v], add=True)

    offt = base + NFULL * C
    pltpu.sync_copy(src_hbm.at[pl.ds(offt, TAIL)], src_t)
    pltpu.sync_copy(dst_hbm.at[pl.ds(offt, TAIL)], dst_t)
    pltpu.async_copy(h_hbm.at[src_t], rows_t, gsem).wait()
    pltpu.sync_copy(rows_t, acc.at[dst_t], add=True)

    plsc.subcore_barrier()

    for k in range(SLAB // C):
        pltpu.sync_copy(acc.at[pl.ds(row0 + k * C, C)], rows_v)
        pltpu.sync_copy(rows_v, out_hbm.at[pl.ds(c * NP + row0 + k * C, C)])


_sc_agg = functools.partial(
    pl.kernel, mesh=_MESH,
    out_type=jax.ShapeDtypeStruct((NC * NP, D), jnp.float32),
    scratch_types=[
        pltpu.VMEM_SHARED((NP, D), jnp.float32),
        pltpu.VMEM((C,), jnp.int32),
        pltpu.VMEM((C,), jnp.int32),
        pltpu.VMEM((C, D), jnp.float32),
        pltpu.VMEM((TAIL,), jnp.int32),
        pltpu.VMEM((TAIL,), jnp.int32),
        pltpu.VMEM((TAIL, D), jnp.float32),
        pltpu.SemaphoreType.DMA,
    ])(_sc_agg_body)


def _dense_body(agg0, agg1, cnt0, cnt1, h, wl, wr, b, out):
    a = agg0[...] + agg1[...]
    cnt = cnt0[...][:, 0:1] + cnt1[...][:, 0:1]
    mean = a * (1.0 / jnp.maximum(cnt, 1.0))
    acc = lax.dot_general(mean, wl[...], (((1,), (1,)), ((), ())),
                          preferred_element_type=jnp.float32)
    acc = acc + lax.dot_general(h[...], wr[...], (((1,), (1,)), ((), ())),
                                preferred_element_type=jnp.float32)
    acc = acc + b[...]
    out[...] = jnp.maximum(acc, 0.0)


def _dense_final_body(agg0, agg1, cnt0, cnt1, h, wl, wr, b, fcw, fcb, out):
    a = agg0[...] + agg1[...]
    cnt = cnt0[...][:, 0:1] + cnt1[...][:, 0:1]
    mean = a * (1.0 / jnp.maximum(cnt, 1.0))
    acc = lax.dot_general(mean, wl[...], (((1,), (1,)), ((), ())),
                          preferred_element_type=jnp.float32)
    acc = acc + lax.dot_general(h[...], wr[...], (((1,), (1,)), ((), ())),
                                preferred_element_type=jnp.float32)
    acc = jnp.maximum(acc + b[...], 0.0)
    out[...] = lax.dot_general(acc, fcw[...], (((1,), (1,)), ((), ())),
                               preferred_element_type=jnp.float32) + fcb[...]


def _row_spec(i):
    return (i, 0)


def _row_spec_hi(i):
    return (i + NP // BM, 0)


def _fixed_spec(i):
    return (0, 0)


_COMMON_SPECS = [
    pl.BlockSpec((BM, D), _row_spec),
    pl.BlockSpec((BM, D), _row_spec_hi),
    pl.BlockSpec((BM, D), _row_spec),
    pl.BlockSpec((BM, D), _row_spec_hi),
    pl.BlockSpec((BM, D), _row_spec),
    pl.BlockSpec((D, D), _fixed_spec),
    pl.BlockSpec((D, D), _fixed_spec),
    pl.BlockSpec((1, D), _fixed_spec),
]

_dense = pl.pallas_call(
    _dense_body,
    grid=(NP // BM,),
    in_specs=_COMMON_SPECS,
    out_specs=pl.BlockSpec((BM, D), _row_spec),
    out_shape=jax.ShapeDtypeStruct((NP, D), jnp.float32),
)

_dense_final = pl.pallas_call(
    _dense_final_body,
    grid=(NP // BM,),
    in_specs=_COMMON_SPECS + [
        pl.BlockSpec((D, D), _fixed_spec),
        pl.BlockSpec((1, D), _fixed_spec),
    ],
    out_specs=pl.BlockSpec((BM, D), _row_spec),
    out_shape=jax.ShapeDtypeStruct((NP, D), jnp.float32),
)


def kernel(x, edge_index, Wl1a, bl1a, Wr1a, Wl1b, bl1b, Wr1b,
           Wl2a, bl2a, Wr2a, Wl2b, bl2b, Wr2b, fc_W, fc_b):
    src = edge_index[0]
    dst = edge_index[1]
    xp = jnp.pad(x, ((0, NP - N), (0, 0)))
    z128 = jnp.zeros((C, D), jnp.float32)
    ones_tbl = jnp.ones((NP, D), jnp.float32)

    cnt = _sc_agg(ones_tbl, src, dst, z128)
    agg = _sc_agg(xp, src, dst, z128)
    h = _dense(agg, agg, cnt, cnt, xp, Wl1a, Wr1a, bl1a.reshape(1, D))
    agg2 = _sc_agg(h, src, dst, z128)
    h = _dense(agg2, agg2, cnt, cnt, h, Wl1b, Wr1b, bl1b.reshape(1, D))
    agg3 = _sc_agg(h, src, dst, z128)
    h = _dense(agg3, agg3, cnt, cnt, h, Wl2a, Wr2a, bl2a.reshape(1, D))
    agg4 = _sc_agg(h, src, dst, z128)
    h = _dense_final(agg4, agg4, cnt, cnt, h, Wl2b, Wr2b, bl2b.reshape(1, D),
                     fc_W, fc_b.reshape(1, D))
    return h[:N]


# speedup vs baseline: 17.4819x; 1.8273x over previous
import functools

import jax
import jax.numpy as jnp
from jax import lax
from jax.experimental import pallas as pl
from jax.experimental.pallas import tpu as pltpu
from jax.experimental.pallas import tpu_sc as plsc

N = 10000
D = 128
E = 320000
NC = 2
NS = 16
NW = NC * NS
NP = 10240
SLAB = NP // NS
EP = E // NW
C = 128
NFULL = EP // C
TAIL = EP - NFULL * C
BM = 256

_MESH = plsc.VectorSubcoreMesh(core_axis_name="c", subcore_axis_name="s")


def _sc_agg_body(h_hbm, src_hbm, dst_hbm, z128_hbm, out_hbm,
                 acc, src4, dst4, rows2, src_t, dst_t, rows_t,
                 gsem0, gsem1, ssem0, ssem1):
    c = lax.axis_index("c")
    s = lax.axis_index("s")
    wid = c * NS + s
    base = wid * EP
    row0 = s * SLAB
    gsem = (gsem0, gsem1)
    ssem = (ssem0, ssem1)

    pltpu.sync_copy(z128_hbm, rows2.at[0])
    for k in range(SLAB // C):
        pltpu.sync_copy(rows2.at[0], acc.at[pl.ds(row0 + k * C, C)])
    plsc.subcore_barrier()

    for b in range(2):
        pltpu.sync_copy(src_hbm.at[pl.ds(base + b * C, C)], src4.at[b])
        pltpu.sync_copy(dst_hbm.at[pl.ds(base + b * C, C)], dst4.at[b])
        pltpu.async_copy(h_hbm.at[src4.at[b]], rows2.at[b], gsem[b])

    @pl.loop(0, NFULL - 2, step=2)
    def chunk_body(i):
        for b in range(2):
            j = i + b
            jn = j + 2
            pltpu.make_async_copy(
                h_hbm.at[pl.ds(0, C)], rows2.at[b], gsem[b]).wait()
            pltpu.async_copy(rows2.at[b], acc.at[dst4.at[j % 4]], ssem[b],
                             add=True)
            pltpu.sync_copy(src_hbm.at[pl.ds(base + jn * C, C)],
                            src4.at[jn % 4])
            pltpu.sync_copy(dst_hbm.at[pl.ds(base + jn * C, C)],
                            dst4.at[jn % 4])
            pltpu.make_async_copy(
                h_hbm.at[pl.ds(0, C)], rows2.at[b], ssem[b]).wait()
            pltpu.async_copy(h_hbm.at[src4.at[jn % 4]], rows2.at[b], gsem[b])

    for b in range(2):
        j = NFULL - 2 + b
        pltpu.make_async_copy(
            h_hbm.at[pl.ds(0, C)], rows2.at[b], gsem[b]).wait()
        pltpu.async_copy(rows2.at[b], acc.at[dst4.at[j % 4]], ssem[b],
                         add=True)

    offt = base + NFULL * C
    pltpu.sync_copy(src_hbm.at[pl.ds(offt, TAIL)], src_t)
    pltpu.sync_copy(dst_hbm.at[pl.ds(offt, TAIL)], dst_t)
    pltpu.async_copy(h_hbm.at[src_t], rows_t, gsem0).wait()
    pltpu.sync_copy(rows_t, acc.at[dst_t], add=True)

    for b in range(2):
        pltpu.make_async_copy(
            h_hbm.at[pl.ds(0, C)], rows2.at[b], ssem[b]).wait()

    plsc.subcore_barrier()

    for k in range(SLAB // C):
        pltpu.sync_copy(acc.at[pl.ds(row0 + k * C, C)], rows2.at[0])
        pltpu.sync_copy(rows2.at[0], out_hbm.at[pl.ds(c * NP + row0 + k * C, C)])


_sc_agg = functools.partial(
    pl.kernel, mesh=_MESH,
    out_type=jax.ShapeDtypeStruct((NC * NP, D), jnp.float32),
    scratch_types=[
        pltpu.VMEM_SHARED((NP, D), jnp.float32),
        pltpu.VMEM((4, C), jnp.int32),
        pltpu.VMEM((4, C), jnp.int32),
        pltpu.VMEM((2, C, D), jnp.float32),
        pltpu.VMEM((TAIL,), jnp.int32),
        pltpu.VMEM((TAIL,), jnp.int32),
        pltpu.VMEM((TAIL, D), jnp.float32),
        pltpu.SemaphoreType.DMA,
        pltpu.SemaphoreType.DMA,
        pltpu.SemaphoreType.DMA,
        pltpu.SemaphoreType.DMA,
    ])(_sc_agg_body)


def _dense_body(agg0, agg1, cnt0, cnt1, h, wl, wr, b, out):
    a = agg0[...] + agg1[...]
    cnt = cnt0[...][:, 0:1] + cnt1[...][:, 0:1]
    mean = a * (1.0 / jnp.maximum(cnt, 1.0))
    acc = lax.dot_general(mean, wl[...], (((1,), (1,)), ((), ())),
                          preferred_element_type=jnp.float32)
    acc = acc + lax.dot_general(h[...], wr[...], (((1,), (1,)), ((), ())),
                                preferred_element_type=jnp.float32)
    acc = acc + b[...]
    out[...] = jnp.maximum(acc, 0.0)


def _dense_final_body(agg0, agg1, cnt0, cnt1, h, wl, wr, b, fcw, fcb, out):
    a = agg0[...] + agg1[...]
    cnt = cnt0[...][:, 0:1] + cnt1[...][:, 0:1]
    mean = a * (1.0 / jnp.maximum(cnt, 1.0))
    acc = lax.dot_general(mean, wl[...], (((1,), (1,)), ((), ())),
                          preferred_element_type=jnp.float32)
    acc = acc + lax.dot_general(h[...], wr[...], (((1,), (1,)), ((), ())),
                                preferred_element_type=jnp.float32)
    acc = jnp.maximum(acc + b[...], 0.0)
    out[...] = lax.dot_general(acc, fcw[...], (((1,), (1,)), ((), ())),
                               preferred_element_type=jnp.float32) + fcb[...]


def _row_spec(i):
    return (i, 0)


def _row_spec_hi(i):
    return (i + NP // BM, 0)


def _fixed_spec(i):
    return (0, 0)


_COMMON_SPECS = [
    pl.BlockSpec((BM, D), _row_spec),
    pl.BlockSpec((BM, D), _row_spec_hi),
    pl.BlockSpec((BM, D), _row_spec),
    pl.BlockSpec((BM, D), _row_spec_hi),
    pl.BlockSpec((BM, D), _row_spec),
    pl.BlockSpec((D, D), _fixed_spec),
    pl.BlockSpec((D, D), _fixed_spec),
    pl.BlockSpec((1, D), _fixed_spec),
]

_dense = pl.pallas_call(
    _dense_body,
    grid=(NP // BM,),
    in_specs=_COMMON_SPECS,
    out_specs=pl.BlockSpec((BM, D), _row_spec),
    out_shape=jax.ShapeDtypeStruct((NP, D), jnp.float32),
)

_dense_final = pl.pallas_call(
    _dense_final_body,
    grid=(NP // BM,),
    in_specs=_COMMON_SPECS + [
        pl.BlockSpec((D, D), _fixed_spec),
        pl.BlockSpec((1, D), _fixed_spec),
    ],
    out_specs=pl.BlockSpec((BM, D), _row_spec),
    out_shape=jax.ShapeDtypeStruct((NP, D), jnp.float32),
)


def kernel(x, edge_index, Wl1a, bl1a, Wr1a, Wl1b, bl1b, Wr1b,
           Wl2a, bl2a, Wr2a, Wl2b, bl2b, Wr2b, fc_W, fc_b):
    src = edge_index[0]
    dst = edge_index[1]
    xp = jnp.pad(x, ((0, NP - N), (0, 0)))
    z128 = jnp.zeros((C, D), jnp.float32)
    ones_tbl = jnp.ones((NP, D), jnp.float32)

    cnt = _sc_agg(ones_tbl, src, dst, z128)
    agg = _sc_agg(xp, src, dst, z128)
    h = _dense(agg, agg, cnt, cnt, xp, Wl1a, Wr1a, bl1a.reshape(1, D))
    agg2 = _sc_agg(h, src, dst, z128)
    h = _dense(agg2, agg2, cnt, cnt, h, Wl1b, Wr1b, bl1b.reshape(1, D))
    agg3 = _sc_agg(h, src, dst, z128)
    h = _dense(agg3, agg3, cnt, cnt, h, Wl2a, Wr2a, bl2a.reshape(1, D))
    agg4 = _sc_agg(h, src, dst, z128)
    h = _dense_final(agg4, agg4, cnt, cnt, h, Wl2b, Wr2b, bl2b.reshape(1, D),
                     fc_W, fc_b.reshape(1, D))
    return h[:N]
